# Initial kernel scaffold; baseline (speedup 1.0000x reference)
import jax
import jax.numpy as jnp
from jax.experimental import pallas as pl


def kernel(x, weight, gamma, beta):
    raise NotImplementedError("write your pallas kernel here")


# speedup vs baseline: 1.3888x; 1.3888x over previous
import functools

import jax
import jax.numpy as jnp
from jax import lax
from jax.experimental import pallas as pl
from jax.experimental.pallas import tpu as pltpu
from jax.experimental.pallas import tpu_sc as plsc

_N = 16384
_K = 8192
_D = 32
_BLK = 256
_NBLK = _N // _BLK
_EPS = 1e-5

_NC = 2
_NS = 16
_NW = _NC * _NS
_BPW = _N // _NW


def _argmin_body(x_blk_ref, x_full_ref, wt_ref, gamma_ref, beta_ref,
                 idx_ref, mean_ref, var_ref, w2_ref):
    i = pl.program_id(0)

    @pl.when(i == 0)
    def _():
        xf = x_full_ref[...]
        mean = jnp.mean(xf, axis=0, keepdims=True)
        var = jnp.mean((xf - mean) ** 2, axis=0, keepdims=True)
        mean_ref[...] = mean
        var_ref[...] = var
        wt = wt_ref[...]
        w2_ref[...] = jnp.sum(wt * wt, axis=0, keepdims=True)

    mean = mean_ref[...]
    var = var_ref[...]
    xb = (x_blk_ref[...] - mean) / jnp.sqrt(var + _EPS) * gamma_ref[...] \
        + beta_ref[...]
    scores = lax.dot_general(
        xb.astype(jnp.bfloat16), wt_ref[...].astype(jnp.bfloat16),
        (((1,), (0,)), ((), ())),
        preferred_element_type=jnp.float32)
    x2 = jnp.sum(xb * xb, axis=1, keepdims=True)
    dist = x2 + w2_ref[...] - 2.0 * scores
    m = jnp.min(dist, axis=1, keepdims=True)
    cand = jnp.where(dist == m,
                     lax.broadcasted_iota(jnp.int32, dist.shape, 1), _K)
    idx_ref[0, 0, :] = jnp.min(cand, axis=1)


_argmin_call = pl.pallas_call(
    _argmin_body,
    grid=(_NBLK,),
    in_specs=[
        pl.BlockSpec((_BLK, _D), lambda i: (i, 0)),
        pl.BlockSpec((_N, _D), lambda i: (0, 0)),
        pl.BlockSpec((_D, _K), lambda i: (0, 0)),
        pl.BlockSpec((1, _D), lambda i: (0, 0)),
        pl.BlockSpec((1, _D), lambda i: (0, 0)),
    ],
    out_specs=pl.BlockSpec((1, 1, _BLK), lambda i: (i, 0, 0)),
    out_shape=jax.ShapeDtypeStruct((_NBLK, 1, _BLK), jnp.int32),
    scratch_shapes=[
        pltpu.VMEM((1, _D), jnp.float32),
        pltpu.VMEM((1, _D), jnp.float32),
        pltpu.VMEM((1, _K), jnp.float32),
    ],
)


@functools.cache
def _make_gather():
    mesh = plsc.VectorSubcoreMesh(
        core_axis_name="c", subcore_axis_name="s",
        num_cores=_NC, num_subcores=_NS)

    @functools.partial(
        pl.kernel, mesh=mesh,
        compiler_params=pltpu.CompilerParams(use_tc_tiling_on_sc=False),
        out_type=jax.ShapeDtypeStruct((_N, _D), jnp.float32),
        scratch_types=[
            pltpu.VMEM((_BPW,), jnp.int32),
            pltpu.VMEM((_BPW, _D), jnp.float32),
            pltpu.SemaphoreType.DMA,
        ],
    )
    def _gather_kernel(table_hbm, idx_hbm, out_hbm, idx_v, rows_v, sem):
        wid = lax.axis_index("s") * _NC + lax.axis_index("c")
        base = wid * _BPW
        pltpu.sync_copy(idx_hbm.at[pl.ds(base, _BPW)], idx_v)
        pltpu.async_copy(table_hbm.at[idx_v], rows_v, sem).wait()
        pltpu.sync_copy(rows_v, out_hbm.at[pl.ds(base, _BPW)])

    return _gather_kernel


def kernel(x, weight, gamma, beta):
    wt = weight.T
    idx3 = _argmin_call(x, x, wt, gamma.reshape(1, _D), beta.reshape(1, _D))
    idx = idx3.reshape(_N)
    out = _make_gather()(weight, idx)
    return out
